# baseline (speedup 1.0000x reference)
import functools

import jax
import jax.numpy as jnp
from jax import lax
from jax.experimental import pallas as pl
from jax.experimental.pallas import tpu as pltpu
from jax.experimental.pallas import tpu_sc as plsc

T = 4096
D = 2048
F = 1024
E = 16
B = 256
NB = 2 * T // B + E
NSR = 2 * T + E * B
NTILES = 32
TPT = T // NTILES
CH = 16
WL = 128
CHC = 8


def _router_body(x_ref, wr_ref, pos1_ref, pos2_ref, w1b_ref, w2b_ref, blk_ref):
    x = x_ref[...]
    wr = wr_ref[...]
    logits = lax.dot_general(x, wr, (((1,), (1,)), ((), ())),
                             preferred_element_type=jnp.float32)
    lmax = jnp.max(logits, axis=1, keepdims=True)
    p = jnp.exp(logits - lmax)
    lane = lax.broadcasted_iota(jnp.int32, (T, E), 1)
    m1 = jnp.max(p, axis=1, keepdims=True)
    a1 = jnp.min(jnp.where(p >= m1, lane, E), axis=1, keepdims=True)
    p2m = jnp.where(lane == a1, -1.0, p)
    m2 = jnp.max(p2m, axis=1, keepdims=True)
    a2 = jnp.min(jnp.where(p2m >= m2, lane, E), axis=1, keepdims=True)
    s = m1 + m2
    w1 = m1 / s
    w2 = m2 / s
    w1b_ref[...] = jnp.broadcast_to(w1, (T, WL))
    w2b_ref[...] = jnp.broadcast_to(w2, (T, WL))

    oh1 = (lane == a1).astype(jnp.float32)
    oh2 = (lane == a2).astype(jnp.float32)
    C = 512
    r0 = lax.broadcasted_iota(jnp.int32, (C, C), 0)
    c0 = lax.broadcasted_iota(jnp.int32, (C, C), 1)
    ltri = (r0 >= c0).astype(jnp.float32)
    run = jnp.zeros((1, E), jnp.float32)
    ranks = []
    for a in range(2):
        oh = oh1 if a == 0 else oh2
        for k in range(T // C):
            blkk = lax.slice(oh, (k * C, 0), ((k + 1) * C, E))
            rr = lax.dot_general(ltri, blkk, (((1,), (0,)), ((), ())),
                                 preferred_element_type=jnp.float32) + run
            ranks.append(rr)
            run = lax.slice(rr, (C - 1, 0), (C, E))
    r1 = jnp.concatenate(ranks[: T // C], axis=0)
    r2 = jnp.concatenate(ranks[T // C:], axis=0)
    cnt = run
    ci = cnt.astype(jnp.int32)
    nb = (ci + (B - 1)) // B
    pc = nb * B
    sl = (lax.broadcasted_iota(jnp.int32, (E, E), 0)
          < lax.broadcasted_iota(jnp.int32, (E, E), 1)).astype(jnp.float32)
    po = lax.dot_general(pc.astype(jnp.float32), sl, (((1,), (0,)), ((), ())),
                         preferred_element_type=jnp.float32).astype(jnp.int32)
    bstart = lax.dot_general(nb.astype(jnp.float32), sl, (((1,), (0,)), ((), ())),
                             preferred_element_type=jnp.float32).astype(jnp.int32)

    def _sel_i32(tab, a):
        return jnp.sum(jnp.where(lane == a, jnp.broadcast_to(tab, (T, E)), 0),
                       axis=1, keepdims=True)

    rank1 = jnp.sum(jnp.where(lane == a1, r1, 0.0), axis=1, keepdims=True)
    rank2 = jnp.sum(jnp.where(lane == a2, r2, 0.0), axis=1, keepdims=True)
    pos1 = _sel_i32(po, a1) + rank1.astype(jnp.int32) - 1
    pos2 = _sel_i32(po, a2) + rank2.astype(jnp.int32) - 1
    pos1_ref[...] = pos1.reshape(T)
    pos2_ref[...] = pos2.reshape(T)

    NBP = 64
    jb = lax.broadcasted_iota(jnp.int32, (NBP, E), 0)
    laneb = lax.broadcasted_iota(jnp.int32, (NBP, E), 1)
    bend = bstart + nb
    be = jnp.sum((jb >= jnp.broadcast_to(bend, (NBP, E))).astype(jnp.int32),
                 axis=1, keepdims=True)
    be = jnp.minimum(be, E - 1)
    total = jnp.sum(nb.astype(jnp.float32), axis=1, keepdims=True).astype(jnp.int32)
    jb0 = lax.slice(jb, (0, 0), (NBP, 1))
    totb = jnp.broadcast_to(total, (NBP, 1))
    belast = jnp.sum(jnp.where(jb0 == totb - 1, be, 0), axis=0, keepdims=True)
    be = jnp.where(jb0 < totb, be, jnp.broadcast_to(belast, (NBP, 1)))
    bsel = jnp.sum(jnp.where(laneb == be, jnp.broadcast_to(bstart, (NBP, E)), 0),
                   axis=1, keepdims=True)
    csel = jnp.sum(jnp.where(laneb == be, jnp.broadcast_to(ci, (NBP, E)), 0),
                   axis=1, keepdims=True)
    valid = jnp.clip(csel - (jb0 - bsel) * B, 0, B)
    valid = jnp.where(jb0 < totb, valid, 0)
    xsidx = jnp.minimum(jb0, totb - 1)
    blk_ref[...] = jnp.concatenate(
        [be.reshape(1, NBP), valid.reshape(1, NBP), xsidx.reshape(1, NBP)], axis=0)


def _router(x, wr):
    return pl.pallas_call(
        _router_body,
        grid=(1,),
        in_specs=[
            pl.BlockSpec((T, D), lambda i: (0, 0)),
            pl.BlockSpec((E, D), lambda i: (0, 0)),
        ],
        out_specs=[
            pl.BlockSpec((T,), lambda i: (0,)),
            pl.BlockSpec((T,), lambda i: (0,)),
            pl.BlockSpec((T, WL), lambda i: (0, 0)),
            pl.BlockSpec((T, WL), lambda i: (0, 0)),
            pl.BlockSpec((3, 64), lambda i: (0, 0)),
        ],
        out_shape=[
            jax.ShapeDtypeStruct((T,), jnp.int32),
            jax.ShapeDtypeStruct((T,), jnp.int32),
            jax.ShapeDtypeStruct((T, WL), jnp.float32),
            jax.ShapeDtypeStruct((T, WL), jnp.float32),
            jax.ShapeDtypeStruct((3, 64), jnp.int32),
        ],
    )(x, wr)


_SC_MESH = plsc.VectorSubcoreMesh(core_axis_name="c", subcore_axis_name="s",
                                  num_cores=2, num_subcores=16)


@functools.partial(
    pl.kernel,
    out_type=(jax.ShapeDtypeStruct((NSR, D), jnp.float32),
              jax.ShapeDtypeStruct((NSR, WL), jnp.float32)),
    mesh=_SC_MESH,
    scratch_types=[
        pltpu.VMEM((TPT,), jnp.int32),
        pltpu.VMEM((TPT,), jnp.int32),
        pltpu.VMEM((CH,), jnp.int32),
        pltpu.VMEM((CH,), jnp.int32),
        pltpu.VMEM((CH,), jnp.int32),
        pltpu.VMEM((CH,), jnp.int32),
        pltpu.VMEM((CH, D), jnp.float32),
        pltpu.VMEM((CH, D), jnp.float32),
        pltpu.VMEM((CH, WL), jnp.float32),
        pltpu.VMEM((CH, WL), jnp.float32),
        pltpu.VMEM((CH, WL), jnp.float32),
        pltpu.VMEM((CH, WL), jnp.float32),
        pltpu.SemaphoreType.DMA,
        pltpu.SemaphoreType.DMA,
    ],
)
def _dispatch(x_hbm, pos1_hbm, pos2_hbm, w1b_hbm, w2b_hbm, xs_hbm, ws_hbm,
              p1_v, p2_v, ia0, ia1, ib0, ib1, xb0, xb1, w10, w11, w20, w21,
              sem_rd, sem_sc):
    ia, ib = [ia0, ia1], [ib0, ib1]
    xb, w1b_, w2b_ = [xb0, xb1], [w10, w11], [w20, w21]
    wid = lax.axis_index("s") * 2 + lax.axis_index("c")
    base = wid * TPT
    pltpu.sync_copy(pos1_hbm.at[pl.ds(base, TPT)], p1_v)
    pltpu.sync_copy(pos2_hbm.at[pl.ds(base, TPT)], p2_v)
    nch = TPT // CH
    rd, sc = {}, {}

    def issue_read(c):
        tb = base + c * CH
        k = c & 1
        rd[c] = (pltpu.async_copy(x_hbm.at[pl.ds(tb, CH)], xb[k], sem_rd),
                 pltpu.async_copy(w1b_hbm.at[pl.ds(tb, CH)], w1b_[k], sem_rd),
                 pltpu.async_copy(w2b_hbm.at[pl.ds(tb, CH)], w2b_[k], sem_rd))

    issue_read(0)
    for c in range(nch):
        k = c & 1
        for d in rd[c]:
            d.wait()
        if c >= 1:
            for d in sc[c - 1]:
                d.wait()
        if c + 1 < nch:
            issue_read(c + 1)
        ia[k][...] = p1_v[pl.ds(c * CH, CH)]
        ib[k][...] = p2_v[pl.ds(c * CH, CH)]
        sc[c] = (pltpu.async_copy(xb[k], xs_hbm.at[ia[k]], sem_sc),
                 pltpu.async_copy(xb[k], xs_hbm.at[ib[k]], sem_sc),
                 pltpu.async_copy(w1b_[k], ws_hbm.at[ia[k]], sem_sc),
                 pltpu.async_copy(w2b_[k], ws_hbm.at[ib[k]], sem_sc))
    for d in sc[nch - 1]:
        d.wait()


def _grouped_body(be_ref, valid_ref, xsidx_ref, xs_ref, ws_ref,
                  wg_ref, wu_ref, wd_ref, out_ref):
    j = pl.program_id(0)
    v = valid_ref[j]

    @pl.when(v > 0)
    def _():
        xb = xs_ref[...]
        g = lax.dot_general(xb, wg_ref[0], (((1,), (1,)), ((), ())),
                            preferred_element_type=jnp.float32)
        u = lax.dot_general(xb, wu_ref[0], (((1,), (1,)), ((), ())),
                            preferred_element_type=jnp.float32)
        w = ws_ref[:, 0:1]
        sig = 1.0 / (1.0 + jnp.exp(-(w * g)))
        h = (w * w * w) * g * u * sig
        row = lax.broadcasted_iota(jnp.int32, (B, 1), 0)
        h = jnp.where(row < v, h, 0.0)
        out_ref[...] = lax.dot_general(h, wd_ref[0], (((1,), (1,)), ((), ())),
                                       preferred_element_type=jnp.float32)


def _grouped(be, valid, xsidx, xs, ws, Wg, Wu, Wd):
    grid_spec = pltpu.PrefetchScalarGridSpec(
        num_scalar_prefetch=3,
        grid=(NB,),
        in_specs=[
            pl.BlockSpec((B, D), lambda j, be, va, xi: (xi[j], 0)),
            pl.BlockSpec((B, WL), lambda j, be, va, xi: (xi[j], 0)),
            pl.BlockSpec((1, F, D), lambda j, be, va, xi: (be[j], 0, 0)),
            pl.BlockSpec((1, F, D), lambda j, be, va, xi: (be[j], 0, 0)),
            pl.BlockSpec((1, D, F), lambda j, be, va, xi: (be[j], 0, 0)),
        ],
        out_specs=pl.BlockSpec((B, D), lambda j, be, va, xi: (xi[j], 0)),
    )
    return pl.pallas_call(
        _grouped_body,
        grid_spec=grid_spec,
        out_shape=jax.ShapeDtypeStruct((NSR, D), jnp.float32),
        compiler_params=pltpu.CompilerParams(
            dimension_semantics=("arbitrary",)),
    )(be, valid, xsidx, xs, ws, Wg, Wu, Wd)


def _shared_body(x_ref, wg_ref, wu_ref, wd_ref, out_ref):
    xb = x_ref[...]
    g = lax.dot_general(xb, wg_ref[...], (((1,), (1,)), ((), ())),
                        preferred_element_type=jnp.float32)
    u = lax.dot_general(xb, wu_ref[...], (((1,), (1,)), ((), ())),
                        preferred_element_type=jnp.float32)
    h = g * (1.0 / (1.0 + jnp.exp(-g))) * u
    out_ref[...] = lax.dot_general(h, wd_ref[...], (((1,), (1,)), ((), ())),
                                   preferred_element_type=jnp.float32)


def _shared(x, Wg_s, Wu_s, Wd_s):
    TB = 256
    return pl.pallas_call(
        _shared_body,
        grid=(T // TB,),
        in_specs=[
            pl.BlockSpec((TB, D), lambda i: (i, 0)),
            pl.BlockSpec((F, D), lambda i: (0, 0)),
            pl.BlockSpec((F, D), lambda i: (0, 0)),
            pl.BlockSpec((D, F), lambda i: (0, 0)),
        ],
        out_specs=pl.BlockSpec((TB, D), lambda i: (i, 0)),
        out_shape=jax.ShapeDtypeStruct((T, D), jnp.float32),
    )(x, Wg_s, Wu_s, Wd_s)


@functools.partial(
    pl.kernel,
    out_type=jax.ShapeDtypeStruct((T, D), jnp.float32),
    mesh=_SC_MESH,
    scratch_types=[
        pltpu.VMEM((TPT,), jnp.int32),
        pltpu.VMEM((TPT,), jnp.int32),
        pltpu.VMEM((CHC,), jnp.int32),
        pltpu.VMEM((CHC,), jnp.int32),
        pltpu.VMEM((CHC,), jnp.int32),
        pltpu.VMEM((CHC,), jnp.int32),
        pltpu.VMEM((CHC, D), jnp.float32),
        pltpu.VMEM((CHC, D), jnp.float32),
        pltpu.VMEM((CHC, D), jnp.float32),
        pltpu.VMEM((CHC, D), jnp.float32),
        pltpu.VMEM((CHC, D), jnp.float32),
        pltpu.VMEM((CHC, D), jnp.float32),
        pltpu.SemaphoreType.DMA,
        pltpu.SemaphoreType.DMA,
    ],
)
def _combine(ys_hbm, sh_hbm, pos1_hbm, pos2_hbm, out_hbm,
             p1_v, p2_v, ia0, ia1, ib0, ib1,
             y10, y11, y20, y21, sb0, sb1, sem_rd, sem_wr):
    ia, ib = [ia0, ia1], [ib0, ib1]
    y1, y2, sb = [y10, y11], [y20, y21], [sb0, sb1]
    wid = lax.axis_index("s") * 2 + lax.axis_index("c")
    base = wid * TPT
    pltpu.sync_copy(pos1_hbm.at[pl.ds(base, TPT)], p1_v)
    pltpu.sync_copy(pos2_hbm.at[pl.ds(base, TPT)], p2_v)
    nch = TPT // CHC
    rd, wr = {}, {}

    def issue(c):
        tb = base + c * CHC
        k = c & 1
        ia[k][...] = p1_v[pl.ds(c * CHC, CHC)]
        ib[k][...] = p2_v[pl.ds(c * CHC, CHC)]
        rd[c] = (pltpu.async_copy(ys_hbm.at[ia[k]], y1[k], sem_rd),
                 pltpu.async_copy(ys_hbm.at[ib[k]], y2[k], sem_rd),
                 pltpu.async_copy(sh_hbm.at[pl.ds(tb, CHC)], sb[k], sem_rd))

    issue(0)
    for c in range(nch):
        k = c & 1
        for d in rd[c]:
            d.wait()
        if c + 1 < nch:
            if c >= 1:
                wr[c - 1].wait()
            issue(c + 1)
        sbk, y1k, y2k = sb[k], y1[k], y2[k]

        @pl.loop(0, D // 16, unroll=4)
        def _addcol(i):
            off = i * 16
            for r in range(CHC):
                sbk[r, pl.ds(off, 16)] = (sbk[r, pl.ds(off, 16)]
                                          + y1k[r, pl.ds(off, 16)]
                                          + y2k[r, pl.ds(off, 16)])
        wr[c] = pltpu.async_copy(sb[k], out_hbm.at[pl.ds(base + c * CHC, CHC)],
                                 sem_wr)
    wr[nch - 1].wait()


def kernel(x, Wr, Wg, Wu, Wd, Wg_s, Wu_s, Wd_s):
    pos1, pos2, w1b, w2b, blk = _router(x, Wr)
    be, valid, xsidx = blk[0], blk[1], blk[2]
    xs, ws = _dispatch(x, pos1, pos2, w1b, w2b)
    shared = _shared(x, Wg_s, Wu_s, Wd_s)
    ys = _grouped(be, valid, xsidx, xs, ws, Wg, Wu, Wd)
    return _combine(ys, shared, pos1, pos2)


# speedup vs baseline: 1.0734x; 1.0734x over previous
import functools

import jax
import jax.numpy as jnp
from jax import lax
from jax.experimental import pallas as pl
from jax.experimental.pallas import tpu as pltpu
from jax.experimental.pallas import tpu_sc as plsc

T = 4096
D = 2048
F = 1024
E = 16
B = 256
NB = 2 * T // B + E
NSR = 2 * T + E * B
NTILES = 32
TPT = T // NTILES
CH = 16
WL = 128
CHC = 8


def _router_body(x_ref, wr_ref, pos1_ref, pos2_ref, w1b_ref, w2b_ref, blk_ref):
    x = x_ref[...]
    wr = wr_ref[...]
    logits = lax.dot_general(x, wr, (((1,), (1,)), ((), ())),
                             preferred_element_type=jnp.float32)
    lmax = jnp.max(logits, axis=1, keepdims=True)
    p = jnp.exp(logits - lmax)
    lane = lax.broadcasted_iota(jnp.int32, (T, E), 1)
    m1 = jnp.max(p, axis=1, keepdims=True)
    a1 = jnp.min(jnp.where(p >= m1, lane, E), axis=1, keepdims=True)
    p2m = jnp.where(lane == a1, -1.0, p)
    m2 = jnp.max(p2m, axis=1, keepdims=True)
    a2 = jnp.min(jnp.where(p2m >= m2, lane, E), axis=1, keepdims=True)
    s = m1 + m2
    w1 = m1 / s
    w2 = m2 / s
    w1b_ref[...] = jnp.broadcast_to(w1, (T, WL))
    w2b_ref[...] = jnp.broadcast_to(w2, (T, WL))

    oh1 = (lane == a1).astype(jnp.float32)
    oh2 = (lane == a2).astype(jnp.float32)
    C = 512
    r0 = lax.broadcasted_iota(jnp.int32, (C, C), 0)
    c0 = lax.broadcasted_iota(jnp.int32, (C, C), 1)
    ltri = (r0 >= c0).astype(jnp.float32)
    run = jnp.zeros((1, E), jnp.float32)
    ranks = []
    for a in range(2):
        oh = oh1 if a == 0 else oh2
        for k in range(T // C):
            blkk = lax.slice(oh, (k * C, 0), ((k + 1) * C, E))
            rr = lax.dot_general(ltri, blkk, (((1,), (0,)), ((), ())),
                                 preferred_element_type=jnp.float32) + run
            ranks.append(rr)
            run = lax.slice(rr, (C - 1, 0), (C, E))
    r1 = jnp.concatenate(ranks[: T // C], axis=0)
    r2 = jnp.concatenate(ranks[T // C:], axis=0)
    cnt = run
    ci = cnt.astype(jnp.int32)
    nb = (ci + (B - 1)) // B
    pc = nb * B
    sl = (lax.broadcasted_iota(jnp.int32, (E, E), 0)
          < lax.broadcasted_iota(jnp.int32, (E, E), 1)).astype(jnp.float32)
    po = lax.dot_general(pc.astype(jnp.float32), sl, (((1,), (0,)), ((), ())),
                         preferred_element_type=jnp.float32).astype(jnp.int32)
    bstart = lax.dot_general(nb.astype(jnp.float32), sl, (((1,), (0,)), ((), ())),
                             preferred_element_type=jnp.float32).astype(jnp.int32)

    def _sel_i32(tab, a):
        return jnp.sum(jnp.where(lane == a, jnp.broadcast_to(tab, (T, E)), 0),
                       axis=1, keepdims=True)

    rank1 = jnp.sum(jnp.where(lane == a1, r1, 0.0), axis=1, keepdims=True)
    rank2 = jnp.sum(jnp.where(lane == a2, r2, 0.0), axis=1, keepdims=True)
    pos1 = _sel_i32(po, a1) + rank1.astype(jnp.int32) - 1
    pos2 = _sel_i32(po, a2) + rank2.astype(jnp.int32) - 1
    pos1_ref[...] = pos1.reshape(T)
    pos2_ref[...] = pos2.reshape(T)

    NBP = 64
    jb = lax.broadcasted_iota(jnp.int32, (NBP, E), 0)
    laneb = lax.broadcasted_iota(jnp.int32, (NBP, E), 1)
    bend = bstart + nb
    be = jnp.sum((jb >= jnp.broadcast_to(bend, (NBP, E))).astype(jnp.int32),
                 axis=1, keepdims=True)
    be = jnp.minimum(be, E - 1)
    total = jnp.sum(nb.astype(jnp.float32), axis=1, keepdims=True).astype(jnp.int32)
    jb0 = lax.slice(jb, (0, 0), (NBP, 1))
    totb = jnp.broadcast_to(total, (NBP, 1))
    belast = jnp.sum(jnp.where(jb0 == totb - 1, be, 0), axis=0, keepdims=True)
    be = jnp.where(jb0 < totb, be, jnp.broadcast_to(belast, (NBP, 1)))
    bsel = jnp.sum(jnp.where(laneb == be, jnp.broadcast_to(bstart, (NBP, E)), 0),
                   axis=1, keepdims=True)
    csel = jnp.sum(jnp.where(laneb == be, jnp.broadcast_to(ci, (NBP, E)), 0),
                   axis=1, keepdims=True)
    valid = jnp.clip(csel - (jb0 - bsel) * B, 0, B)
    valid = jnp.where(jb0 < totb, valid, 0)
    xsidx = jnp.minimum(jb0, totb - 1)
    blk_ref[...] = jnp.concatenate(
        [be.reshape(1, NBP), valid.reshape(1, NBP), xsidx.reshape(1, NBP)], axis=0)


def _router(x, wr):
    return pl.pallas_call(
        _router_body,
        grid=(1,),
        in_specs=[
            pl.BlockSpec((T, D), lambda i: (0, 0)),
            pl.BlockSpec((E, D), lambda i: (0, 0)),
        ],
        out_specs=[
            pl.BlockSpec((T,), lambda i: (0,)),
            pl.BlockSpec((T,), lambda i: (0,)),
            pl.BlockSpec((T, WL), lambda i: (0, 0)),
            pl.BlockSpec((T, WL), lambda i: (0, 0)),
            pl.BlockSpec((3, 64), lambda i: (0, 0)),
        ],
        out_shape=[
            jax.ShapeDtypeStruct((T,), jnp.int32),
            jax.ShapeDtypeStruct((T,), jnp.int32),
            jax.ShapeDtypeStruct((T, WL), jnp.float32),
            jax.ShapeDtypeStruct((T, WL), jnp.float32),
            jax.ShapeDtypeStruct((3, 64), jnp.int32),
        ],
    )(x, wr)


_SC_MESH = plsc.VectorSubcoreMesh(core_axis_name="c", subcore_axis_name="s",
                                  num_cores=2, num_subcores=16)


@functools.partial(
    pl.kernel,
    out_type=(jax.ShapeDtypeStruct((NSR, D), jnp.float32),
              jax.ShapeDtypeStruct((NSR, WL), jnp.float32)),
    mesh=_SC_MESH,
    scratch_types=[
        pltpu.VMEM((TPT,), jnp.int32),
        pltpu.VMEM((TPT,), jnp.int32),
        pltpu.VMEM((CH,), jnp.int32),
        pltpu.VMEM((CH,), jnp.int32),
        pltpu.VMEM((CH,), jnp.int32),
        pltpu.VMEM((CH,), jnp.int32),
        pltpu.VMEM((CH, D), jnp.float32),
        pltpu.VMEM((CH, D), jnp.float32),
        pltpu.VMEM((CH, WL), jnp.float32),
        pltpu.VMEM((CH, WL), jnp.float32),
        pltpu.VMEM((CH, WL), jnp.float32),
        pltpu.VMEM((CH, WL), jnp.float32),
        pltpu.SemaphoreType.DMA,
        pltpu.SemaphoreType.DMA,
    ],
)
def _dispatch(x_hbm, pos1_hbm, pos2_hbm, w1b_hbm, w2b_hbm, xs_hbm, ws_hbm,
              p1_v, p2_v, ia0, ia1, ib0, ib1, xb0, xb1, w10, w11, w20, w21,
              sem_rd, sem_sc):
    ia, ib = [ia0, ia1], [ib0, ib1]
    xb, w1b_, w2b_ = [xb0, xb1], [w10, w11], [w20, w21]
    wid = lax.axis_index("s") * 2 + lax.axis_index("c")
    base = wid * TPT
    pltpu.sync_copy(pos1_hbm.at[pl.ds(base, TPT)], p1_v)
    pltpu.sync_copy(pos2_hbm.at[pl.ds(base, TPT)], p2_v)
    nch = TPT // CH
    rd, sc = {}, {}

    def issue_read(c):
        tb = base + c * CH
        k = c & 1
        rd[c] = (pltpu.async_copy(x_hbm.at[pl.ds(tb, CH)], xb[k], sem_rd),
                 pltpu.async_copy(w1b_hbm.at[pl.ds(tb, CH)], w1b_[k], sem_rd),
                 pltpu.async_copy(w2b_hbm.at[pl.ds(tb, CH)], w2b_[k], sem_rd))

    issue_read(0)
    for c in range(nch):
        k = c & 1
        for d in rd[c]:
            d.wait()
        if c >= 1:
            for d in sc[c - 1]:
                d.wait()
        if c + 1 < nch:
            issue_read(c + 1)
        ia[k][...] = p1_v[pl.ds(c * CH, CH)]
        ib[k][...] = p2_v[pl.ds(c * CH, CH)]
        sc[c] = (pltpu.async_copy(xb[k], xs_hbm.at[ia[k]], sem_sc),
                 pltpu.async_copy(xb[k], xs_hbm.at[ib[k]], sem_sc),
                 pltpu.async_copy(w1b_[k], ws_hbm.at[ia[k]], sem_sc),
                 pltpu.async_copy(w2b_[k], ws_hbm.at[ib[k]], sem_sc))
    for d in sc[nch - 1]:
        d.wait()


def _grouped_body(be_ref, valid_ref, xsidx_ref, xs_ref, ws_ref,
                  wg_ref, wu_ref, wd_ref, out_ref):
    j = pl.program_id(0)
    v = valid_ref[j]

    @pl.when(v > 0)
    def _():
        xb = xs_ref[...]
        g = lax.dot_general(xb, wg_ref[0], (((1,), (1,)), ((), ())),
                            preferred_element_type=jnp.float32)
        u = lax.dot_general(xb, wu_ref[0], (((1,), (1,)), ((), ())),
                            preferred_element_type=jnp.float32)
        w = ws_ref[:, 0:1]
        sig = 1.0 / (1.0 + jnp.exp(-(w * g)))
        h = (w * w * w) * g * u * sig
        row = lax.broadcasted_iota(jnp.int32, (B, 1), 0)
        h = jnp.where(row < v, h, 0.0)
        out_ref[...] = lax.dot_general(h, wd_ref[0], (((1,), (1,)), ((), ())),
                                       preferred_element_type=jnp.float32)


def _grouped(be, valid, xsidx, xs, ws, Wg, Wu, Wd):
    grid_spec = pltpu.PrefetchScalarGridSpec(
        num_scalar_prefetch=3,
        grid=(NB,),
        in_specs=[
            pl.BlockSpec((B, D), lambda j, be, va, xi: (xi[j], 0)),
            pl.BlockSpec((B, WL), lambda j, be, va, xi: (xi[j], 0)),
            pl.BlockSpec((1, F, D), lambda j, be, va, xi: (be[j], 0, 0)),
            pl.BlockSpec((1, F, D), lambda j, be, va, xi: (be[j], 0, 0)),
            pl.BlockSpec((1, D, F), lambda j, be, va, xi: (be[j], 0, 0)),
        ],
        out_specs=pl.BlockSpec((B, D), lambda j, be, va, xi: (xi[j], 0)),
    )
    return pl.pallas_call(
        _grouped_body,
        grid_spec=grid_spec,
        out_shape=jax.ShapeDtypeStruct((NSR, D), jnp.float32),
        compiler_params=pltpu.CompilerParams(
            dimension_semantics=("arbitrary",)),
    )(be, valid, xsidx, xs, ws, Wg, Wu, Wd)


def _shared_body(x_ref, wg_ref, wu_ref, wd_ref, out_ref):
    xb = x_ref[...]
    g = lax.dot_general(xb, wg_ref[...], (((1,), (1,)), ((), ())),
                        preferred_element_type=jnp.float32)
    u = lax.dot_general(xb, wu_ref[...], (((1,), (1,)), ((), ())),
                        preferred_element_type=jnp.float32)
    h = g * (1.0 / (1.0 + jnp.exp(-g))) * u
    out_ref[...] = lax.dot_general(h, wd_ref[...], (((1,), (1,)), ((), ())),
                                   preferred_element_type=jnp.float32)


def _shared(x, Wg_s, Wu_s, Wd_s):
    TB = 256
    return pl.pallas_call(
        _shared_body,
        grid=(T // TB,),
        in_specs=[
            pl.BlockSpec((TB, D), lambda i: (i, 0)),
            pl.BlockSpec((F, D), lambda i: (0, 0)),
            pl.BlockSpec((F, D), lambda i: (0, 0)),
            pl.BlockSpec((D, F), lambda i: (0, 0)),
        ],
        out_specs=pl.BlockSpec((TB, D), lambda i: (i, 0)),
        out_shape=jax.ShapeDtypeStruct((T, D), jnp.float32),
    )(x, Wg_s, Wu_s, Wd_s)


@functools.partial(
    pl.kernel,
    out_type=jax.ShapeDtypeStruct((T, D), jnp.float32),
    mesh=_SC_MESH,
    scratch_types=[
        pltpu.VMEM((TPT,), jnp.int32),
        pltpu.VMEM((TPT,), jnp.int32),
        pltpu.VMEM((CHC,), jnp.int32),
        pltpu.VMEM((CHC,), jnp.int32),
        pltpu.VMEM((CHC,), jnp.int32),
        pltpu.VMEM((CHC,), jnp.int32),
        pltpu.VMEM((CHC, D), jnp.float32),
        pltpu.VMEM((CHC, D), jnp.float32),
        pltpu.VMEM((CHC, D), jnp.float32),
        pltpu.VMEM((CHC, D), jnp.float32),
        pltpu.VMEM((CHC, D), jnp.float32),
        pltpu.VMEM((CHC, D), jnp.float32),
        pltpu.SemaphoreType.DMA,
        pltpu.SemaphoreType.DMA,
    ],
)
def _combine(ys_hbm, sh_hbm, pos1_hbm, pos2_hbm, out_hbm,
             p1_v, p2_v, ia0, ia1, ib0, ib1,
             y10, y11, y20, y21, sb0, sb1, sem_rd, sem_wr):
    ia, ib = [ia0, ia1], [ib0, ib1]
    y1, y2, sb = [y10, y11], [y20, y21], [sb0, sb1]
    wid = lax.axis_index("s") * 2 + lax.axis_index("c")
    base = wid * TPT
    pltpu.sync_copy(pos1_hbm.at[pl.ds(base, TPT)], p1_v)
    pltpu.sync_copy(pos2_hbm.at[pl.ds(base, TPT)], p2_v)
    nch = TPT // CHC
    rd, wr = {}, {}

    def issue(c):
        tb = base + c * CHC
        k = c & 1
        ia[k][...] = p1_v[pl.ds(c * CHC, CHC)]
        ib[k][...] = p2_v[pl.ds(c * CHC, CHC)]
        rd[c] = (pltpu.async_copy(ys_hbm.at[ia[k]], y1[k], sem_rd),
                 pltpu.async_copy(ys_hbm.at[ib[k]], y2[k], sem_rd),
                 pltpu.async_copy(sh_hbm.at[pl.ds(tb, CHC)], sb[k], sem_rd))

    issue(0)
    for c in range(nch):
        k = c & 1
        for d in rd[c]:
            d.wait()
        if c + 1 < nch:
            if c >= 1:
                wr[c - 1].wait()
            issue(c + 1)
        sbk, y1k, y2k = sb[k], y1[k], y2[k]

        @pl.loop(0, D // 16)
        def _addcol(i):
            off = i * 16
            for r in range(CHC):
                sbk[r, pl.ds(off, 16)] = (sbk[r, pl.ds(off, 16)]
                                          + y1k[r, pl.ds(off, 16)]
                                          + y2k[r, pl.ds(off, 16)])
        wr[c] = pltpu.async_copy(sb[k], out_hbm.at[pl.ds(base + c * CHC, CHC)],
                                 sem_wr)
    wr[nch - 1].wait()


def kernel(x, Wr, Wg, Wu, Wd, Wg_s, Wu_s, Wd_s):
    pos1, pos2, w1b, w2b, blk = _router(x, Wr)
    be, valid, xsidx = blk[0], blk[1], blk[2]
    xs, ws = _dispatch(x, pos1, pos2, w1b, w2b)
    shared = _shared(x, Wg_s, Wu_s, Wd_s)
    ys = _grouped(be, valid, xsidx, xs, ws, Wg, Wu, Wd)
    return _combine(ys, shared, pos1, pos2)
